# baseline (speedup 1.0000x reference)
import jax
import jax.numpy as jnp
from jax.experimental import pallas as pl
from jax.experimental.pallas import tpu as pltpu


def _proj_body(x_ref, w_ref, b_ref, o_ref):
    o_ref[...] = (
        jnp.dot(x_ref[...], w_ref[...], preferred_element_type=jnp.float32)
        + b_ref[...]
    )


def _proj(x2d, W, b):
    M, F = x2d.shape
    Fo = W.shape[1]
    BM = 1000
    return pl.pallas_call(
        _proj_body,
        grid=(M // BM,),
        in_specs=[
            pl.BlockSpec((BM, F), lambda i: (i, 0)),
            pl.BlockSpec((F, Fo), lambda i: (0, 0)),
            pl.BlockSpec((1, Fo), lambda i: (0, 0)),
        ],
        out_specs=pl.BlockSpec((BM, Fo), lambda i: (i, 0)),
        out_shape=jax.ShapeDtypeStruct((M, Fo), jnp.float32),
    )(x2d, W, b.reshape(1, Fo))


def _norms(row, col, edge_weight, N):
    deg_out = jnp.zeros((N,), dtype=edge_weight.dtype).at[row].add(edge_weight)
    deg_in = jnp.zeros((N,), dtype=edge_weight.dtype).at[col].add(edge_weight)
    deg_out_inv = jnp.where(deg_out > 0, 1.0 / jnp.where(deg_out > 0, deg_out, 1.0), 0.0)
    deg_in_inv = jnp.where(deg_in > 0, 1.0 / jnp.where(deg_in > 0, deg_in, 1.0), 0.0)
    norm_out = deg_out_inv[row] * edge_weight
    norm_in = deg_in_inv[col] * edge_weight
    return norm_out, norm_in


def _dconv(X, row, col, norm_out, norm_in, weight, bias):
    K = weight.shape[1]

    def prop_fwd(v):
        return jnp.zeros_like(v).at[row].add(norm_out[:, None] * v[col])

    def prop_bwd(v):
        return jnp.zeros_like(v).at[col].add(norm_in[:, None] * v[row])

    H = X @ weight[0, 0] + X @ weight[1, 0]
    Tx1_o = X
    Tx1_i = X
    if K > 1:
        Tx1_o = prop_fwd(X)
        Tx1_i = prop_bwd(X)
        H = H + Tx1_o @ weight[0, 1] + Tx1_i @ weight[1, 1]
    Tx0_o = X
    Tx0_i = X
    for k in range(2, K):
        Tx2_o = 2.0 * prop_fwd(Tx1_o) - Tx0_o
        Tx2_i = 2.0 * prop_bwd(Tx1_i) - Tx0_i
        H = H + Tx2_o @ weight[0, k] + Tx2_i @ weight[1, k]
        Tx0_o, Tx1_o = Tx1_o, Tx2_o
        Tx0_i, Tx1_i = Tx1_i, Tx2_i
    return H + bias


def _cell(X, H, row, col, norm_out, norm_in, Wz, bz, Wr, br, Wh, bh):
    XH = jnp.concatenate([X, H], axis=-1)
    Z = jax.nn.sigmoid(_dconv(XH, row, col, norm_out, norm_in, Wz, bz))
    R = jax.nn.sigmoid(_dconv(XH, row, col, norm_out, norm_in, Wr, br))
    XRH = jnp.concatenate([X, R * H], axis=-1)
    H_tilde = jnp.tanh(_dconv(XRH, row, col, norm_out, norm_in, Wh, bh))
    return Z * H + (1.0 - Z) * H_tilde


def kernel(x, edge_index, edge_weight, Wz, bz, Wr, br, Wh, bh, W_lin, b_lin):
    B, N, Fin, P = x.shape
    Fout = Wz.shape[-1]
    row, col = edge_index[0], edge_index[1]
    norm_out, norm_in = _norms(row, col, edge_weight, N)
    xs = jnp.transpose(x, (3, 1, 0, 2)).reshape(P, N, B * Fin)
    H = jnp.zeros((N, Fout), dtype=x.dtype)
    hs = []
    for t in range(P):
        H = _cell(xs[t], H, row, col, norm_out, norm_in, Wz, bz, Wr, br, Wh, bh)
        H = jax.nn.relu(H)
        hs.append(H)
    Hs = jnp.stack(hs, axis=0)
    out = _proj(Hs.reshape(P * N, Fout), W_lin, b_lin).reshape(P, N, Fin)
    out_seq = jnp.transpose(out[:, :, None, :], (2, 1, 3, 0))
    return out_seq, H


# speedup vs baseline: 1.6918x; 1.6918x over previous
import functools

import jax
import jax.numpy as jnp
from jax import lax
from jax.experimental import pallas as pl
from jax.experimental.pallas import tpu as pltpu
from jax.experimental.pallas import tpu_sc as plsc

_NC = 2
_NS = 16
_NW = _NC * _NS
_F = 128
_CHUNK = 400


_FH = _F // 2


def _prop_body(n_pad, table2, src, dst, scale, zeros, out,
               src_v, dst_v, scale_v, rows_v, acc, sem):
    c = lax.axis_index("c")
    s = lax.axis_index("s")
    rpt = n_pad // _NS

    pltpu.sync_copy(zeros.at[pl.ds(s * rpt, rpt)], acc.at[pl.ds(s * rpt, rpt)])
    plsc.subcore_barrier()

    e_total = src.shape[0]
    e_per_w = e_total // _NS
    n_chunks = e_per_w // _CHUNK

    def chunk_body(i, carry):
        base = s * e_per_w + i * _CHUNK
        pltpu.sync_copy(src.at[pl.ds(base, _CHUNK)], src_v)
        pltpu.sync_copy(dst.at[pl.ds(base, _CHUNK)], dst_v)
        pltpu.sync_copy(scale.at[pl.ds(base, _CHUNK)], scale_v)
        for j in range(_CHUNK // 16):
            sl = pl.ds(j * 16, 16)
            src_v[sl] = src_v[sl] * 2 + c
        pltpu.async_copy(table2.at[src_v], rows_v, sem).wait()

        def scale_body(g, c2):
            sv16 = scale_v[pl.ds(g * 16, 16)]
            for l in range(16):
                bl = lax.gather(
                    sv16, jnp.full((16, 1), l, jnp.int32),
                    lax.GatherDimensionNumbers(offset_dims=(),
                                               collapsed_slice_dims=(0,),
                                               start_index_map=(0,)),
                    (1,), mode=lax.GatherScatterMode.PROMISE_IN_BOUNDS)
                e = g * 16 + l
                for j in range(_FH // 16):
                    sl = pl.ds(j * 16, 16)
                    rows_v[e, sl] = rows_v[e, sl] * bl
            return c2

        lax.fori_loop(0, _CHUNK // 16, scale_body, 0)
        pltpu.sync_copy(rows_v, acc.at[dst_v], add=True)
        return carry

    lax.fori_loop(0, n_chunks, chunk_body, 0)
    plsc.subcore_barrier()
    pltpu.sync_copy(acc.at[pl.ds(s * rpt, rpt)],
                    out.at[c, pl.ds(s * rpt, rpt)])


def _sc_prop(table, src, dst, scale, zeros):
    n = table.shape[0]
    n_pad = (n + _NS * 8 - 1) // (_NS * 8) * (_NS * 8)
    mesh = plsc.VectorSubcoreMesh(core_axis_name="c", subcore_axis_name="s")
    kern = functools.partial(
        pl.kernel,
        mesh=mesh,
        compiler_params=pltpu.CompilerParams(use_tc_tiling_on_sc=False),
        out_type=jax.ShapeDtypeStruct((_NC, n_pad, _FH), jnp.float32),
        scratch_types=[
            pltpu.VMEM((_CHUNK,), jnp.int32),
            pltpu.VMEM((_CHUNK,), jnp.int32),
            pltpu.VMEM((_CHUNK,), jnp.float32),
            pltpu.VMEM((_CHUNK, _FH), jnp.float32),
            pltpu.VMEM_SHARED((n_pad, _FH), jnp.float32),
            pltpu.SemaphoreType.DMA,
        ],
    )(functools.partial(_prop_body, n_pad))
    p = kern(table.reshape(2 * n, _FH), src, dst, scale, zeros)
    return p[:, :n].transpose(1, 0, 2).reshape(n, _F)


def _proj_matmul_body(x_ref, w_ref, b_ref, o_ref):
    o_ref[...] = (
        jnp.dot(x_ref[...], w_ref[...], preferred_element_type=jnp.float32)
        + b_ref[...]
    )


def _proj(x2d, W, b):
    M, F = x2d.shape
    Fo = W.shape[1]
    BM = 1000
    return pl.pallas_call(
        _proj_matmul_body,
        grid=(M // BM,),
        in_specs=[
            pl.BlockSpec((BM, F), lambda i: (i, 0)),
            pl.BlockSpec((F, Fo), lambda i: (0, 0)),
            pl.BlockSpec((1, Fo), lambda i: (0, 0)),
        ],
        out_specs=pl.BlockSpec((BM, Fo), lambda i: (i, 0)),
        out_shape=jax.ShapeDtypeStruct((M, Fo), jnp.float32),
    )(x2d, W, b.reshape(1, Fo))


def kernel(x, edge_index, edge_weight, Wz, bz, Wr, br, Wh, bh, W_lin, b_lin):
    B, N, Fin, P = x.shape
    Fout = Wz.shape[-1]
    row, col = edge_index[0], edge_index[1]

    deg_out = jnp.zeros((N,), dtype=edge_weight.dtype).at[row].add(edge_weight)
    deg_in = jnp.zeros((N,), dtype=edge_weight.dtype).at[col].add(edge_weight)
    deg_out_inv = jnp.where(deg_out > 0, 1.0 / jnp.where(deg_out > 0, deg_out, 1.0), 0.0)
    deg_in_inv = jnp.where(deg_in > 0, 1.0 / jnp.where(deg_in > 0, deg_in, 1.0), 0.0)
    norm_out = deg_out_inv[row] * edge_weight
    norm_in = deg_in_inv[col] * edge_weight

    n_pad = (N + _NS * 8 - 1) // (_NS * 8) * (_NS * 8)
    zeros = jnp.zeros((n_pad, _FH), jnp.float32)

    def pf(v):
        return _sc_prop(v, col, row, norm_out, zeros)

    def pb(v):
        return _sc_prop(v, row, col, norm_in, zeros)

    def basis(S):
        Sf = pf(S)
        Sb = pb(S)
        Sff = 2.0 * pf(Sf) - S
        Sbb = 2.0 * pb(Sb) - S
        return Sf, Sb, Sff, Sbb

    xs = jnp.transpose(x, (3, 1, 0, 2)).reshape(P, N, B * Fin)
    XBs = [basis(xs[t]) for t in range(P)]

    def dconv_like(XH, T1o, T1i, T2o, T2i, W, b):
        Hm = XH @ W[0, 0] + XH @ W[1, 0]
        Hm = Hm + T1o @ W[0, 1] + T1i @ W[1, 1]
        Hm = Hm + T2o @ W[0, 2] + T2i @ W[1, 2]
        return Hm + b

    H = jnp.zeros((N, Fout), jnp.float32)
    hs = []
    for t in range(P):
        X = xs[t]
        Xf, Xb, Xff, Xbb = XBs[t]
        Hf, Hb, Hff, Hbb = basis(H)
        XH = jnp.concatenate([X, H], axis=-1)
        T1o = jnp.concatenate([Xf, Hf], axis=-1)
        T1i = jnp.concatenate([Xb, Hb], axis=-1)
        T2o = jnp.concatenate([Xff, Hff], axis=-1)
        T2i = jnp.concatenate([Xbb, Hbb], axis=-1)
        Z = jax.nn.sigmoid(dconv_like(XH, T1o, T1i, T2o, T2i, Wz, bz))
        Rg = jax.nn.sigmoid(dconv_like(XH, T1o, T1i, T2o, T2i, Wr, br))
        G = Rg * H
        Gf, Gb, Gff, Gbb = basis(G)
        XG = jnp.concatenate([X, G], axis=-1)
        U1o = jnp.concatenate([Xf, Gf], axis=-1)
        U1i = jnp.concatenate([Xb, Gb], axis=-1)
        U2o = jnp.concatenate([Xff, Gff], axis=-1)
        U2i = jnp.concatenate([Xbb, Gbb], axis=-1)
        Ht = jnp.tanh(dconv_like(XG, U1o, U1i, U2o, U2i, Wh, bh))
        H = jax.nn.relu(Z * H + (1.0 - Z) * Ht)
        hs.append(H)

    Hs = jnp.stack(hs, axis=0)
    out = _proj(Hs.reshape(P * N, Fout), W_lin, b_lin).reshape(P, N, Fin)
    out_seq = jnp.transpose(out[:, :, None, :], (2, 1, 3, 0))
    return out_seq, H


# speedup vs baseline: 2.2348x; 1.3209x over previous
import functools

import jax
import jax.numpy as jnp
from jax import lax
from jax.experimental import pallas as pl
from jax.experimental.pallas import tpu as pltpu
from jax.experimental.pallas import tpu_sc as plsc

_NC = 2
_NS = 16
_NW = _NC * _NS
_F = 128
_CHUNK = 400
_PHASE_E = 4000


_FH = _F // 2


def _lane_bcast(v16, l):
    return lax.gather(
        v16, jnp.full((16, 1), l, jnp.int32),
        lax.GatherDimensionNumbers(offset_dims=(), collapsed_slice_dims=(0,),
                                   start_index_map=(0,)),
        (1,), mode=lax.GatherScatterMode.PROMISE_IN_BOUNDS)


def _prop_body(n_pad, table2, src, dst, scale, out,
               src_a, dst_a, scale_a, rows0, rows1, acc,
               sem0, sem1, ssem0, ssem1):
    c = lax.axis_index("c")
    s = lax.axis_index("s")
    rpt = n_pad // _NS
    e_per_w = src.shape[0] // _NS
    base_w = s * e_per_w
    rows = (rows0, rows1)
    sems = (sem0, sem1)
    ssems = (ssem0, ssem1)

    zvec = jnp.zeros((16,), jnp.float32)

    def zf(r, carry):
        for j in range(_FH // 16):
            rows0[r, pl.ds(j * 16, 16)] = zvec
        return carry

    lax.fori_loop(0, _CHUNK, zf, 0)
    pltpu.sync_copy(rows0, acc.at[pl.ds(s * rpt, _CHUNK)])
    pltpu.sync_copy(rows0.at[pl.ds(0, rpt - _CHUNK)],
                    acc.at[pl.ds(s * rpt + _CHUNK, rpt - _CHUNK)])
    plsc.subcore_barrier()

    def issue_gather(i, b):
        pltpu.async_copy(table2.at[src_a.at[pl.ds(i * _CHUNK, _CHUNK)]],
                         rows[b], sems[b])

    def wait_gather(b):
        pltpu.make_async_copy(table2.at[pl.ds(0, _CHUNK)], rows[b],
                              sems[b]).wait()

    def issue_scatter(i, b):
        pltpu.async_copy(rows[b],
                         acc.at[dst_a.at[pl.ds(i * _CHUNK, _CHUNK)]],
                         ssems[b], add=True)

    def drain_scatter(b):
        pltpu.make_async_copy(table2.at[pl.ds(0, _CHUNK)], rows[b],
                              ssems[b]).wait()

    def scale_rows(i, b):
        def grp(g, carry):
            e0 = g * 16
            sv16 = scale_a[pl.ds(i * _CHUNK + e0, 16)]
            for l in range(16):
                bl = _lane_bcast(sv16, l)
                for j in range(_FH // 16):
                    sl = pl.ds(j * 16, 16)
                    rows[b][e0 + l, sl] = rows[b][e0 + l, sl] * bl
            return carry

        lax.fori_loop(0, _CHUNK // 16, grp, 0)

    n_ph_pairs = _PHASE_E // _CHUNK // 2

    def phase_body(h, carry):
        pbase = base_w + h * _PHASE_E
        pltpu.sync_copy(src.at[pl.ds(pbase, _PHASE_E)], src_a)
        pltpu.sync_copy(dst.at[pl.ds(pbase, _PHASE_E)], dst_a)
        pltpu.sync_copy(scale.at[pl.ds(pbase, _PHASE_E)], scale_a)

        def xf(g, carry2):
            sl = pl.ds(g * 16, 16)
            src_a[sl] = src_a[sl] * 2 + c
            return carry2

        lax.fori_loop(0, _PHASE_E // 16, xf, 0)
        issue_gather(0, 0)

        def pair_body(k, carry2):
            ia, ib = 2 * k, 2 * k + 1
            wait_gather(0)

            @pl.when(k > 0)
            def _drain_b():
                drain_scatter(1)

            issue_gather(ib, 1)
            scale_rows(ia, 0)
            issue_scatter(ia, 0)
            wait_gather(1)
            scale_rows(ib, 1)
            issue_scatter(ib, 1)

            @pl.when(k < n_ph_pairs - 1)
            def _next_a():
                drain_scatter(0)
                issue_gather(ib + 1, 0)

            return carry2

        lax.fori_loop(0, n_ph_pairs, pair_body, 0)
        drain_scatter(0)
        drain_scatter(1)
        return carry

    lax.fori_loop(0, e_per_w // _PHASE_E, phase_body, 0)
    plsc.subcore_barrier()
    pltpu.sync_copy(acc.at[pl.ds(s * rpt, rpt)],
                    out.at[c, pl.ds(s * rpt, rpt)])


def _sc_prop(table, src, dst, scale, zeros):
    del zeros
    n = table.shape[0]
    n_pad = (n + _NS * 8 - 1) // (_NS * 8) * (_NS * 8)
    e_per_w = src.shape[0] // _NS
    mesh = plsc.VectorSubcoreMesh(core_axis_name="c", subcore_axis_name="s")
    kern = functools.partial(
        pl.kernel,
        mesh=mesh,
        compiler_params=pltpu.CompilerParams(use_tc_tiling_on_sc=False),
        out_type=jax.ShapeDtypeStruct((_NC, n_pad, _FH), jnp.float32),
        scratch_types=[
            pltpu.VMEM((_PHASE_E,), jnp.int32),
            pltpu.VMEM((_PHASE_E,), jnp.int32),
            pltpu.VMEM((_PHASE_E,), jnp.float32),
            pltpu.VMEM((_CHUNK, _FH), jnp.float32),
            pltpu.VMEM((_CHUNK, _FH), jnp.float32),
            pltpu.VMEM_SHARED((n_pad, _FH), jnp.float32),
            pltpu.SemaphoreType.DMA,
            pltpu.SemaphoreType.DMA,
            pltpu.SemaphoreType.DMA,
            pltpu.SemaphoreType.DMA,
        ],
    )(functools.partial(_prop_body, n_pad))
    p = kern(table.reshape(2 * n, _FH), src, dst, scale)
    return p[:, :n].transpose(1, 0, 2).reshape(n, _F)


def _proj_matmul_body(x_ref, w_ref, b_ref, o_ref):
    o_ref[...] = (
        jnp.dot(x_ref[...], w_ref[...], preferred_element_type=jnp.float32)
        + b_ref[...]
    )


def _proj(x2d, W, b):
    M, F = x2d.shape
    Fo = W.shape[1]
    BM = 1000
    return pl.pallas_call(
        _proj_matmul_body,
        grid=(M // BM,),
        in_specs=[
            pl.BlockSpec((BM, F), lambda i: (i, 0)),
            pl.BlockSpec((F, Fo), lambda i: (0, 0)),
            pl.BlockSpec((1, Fo), lambda i: (0, 0)),
        ],
        out_specs=pl.BlockSpec((BM, Fo), lambda i: (i, 0)),
        out_shape=jax.ShapeDtypeStruct((M, Fo), jnp.float32),
    )(x2d, W, b.reshape(1, Fo))


def kernel(x, edge_index, edge_weight, Wz, bz, Wr, br, Wh, bh, W_lin, b_lin):
    B, N, Fin, P = x.shape
    Fout = Wz.shape[-1]
    row, col = edge_index[0], edge_index[1]

    deg_out = jnp.zeros((N,), dtype=edge_weight.dtype).at[row].add(edge_weight)
    deg_in = jnp.zeros((N,), dtype=edge_weight.dtype).at[col].add(edge_weight)
    deg_out_inv = jnp.where(deg_out > 0, 1.0 / jnp.where(deg_out > 0, deg_out, 1.0), 0.0)
    deg_in_inv = jnp.where(deg_in > 0, 1.0 / jnp.where(deg_in > 0, deg_in, 1.0), 0.0)
    norm_out = deg_out_inv[row] * edge_weight
    norm_in = deg_in_inv[col] * edge_weight

    n_pad = (N + _NS * 8 - 1) // (_NS * 8) * (_NS * 8)
    zeros = jnp.zeros((n_pad, _FH), jnp.float32)

    def pf(v):
        return _sc_prop(v, col, row, norm_out, zeros)

    def pb(v):
        return _sc_prop(v, row, col, norm_in, zeros)

    def basis(S):
        Sf = pf(S)
        Sb = pb(S)
        Sff = 2.0 * pf(Sf) - S
        Sbb = 2.0 * pb(Sb) - S
        return Sf, Sb, Sff, Sbb

    xs = jnp.transpose(x, (3, 1, 0, 2)).reshape(P, N, B * Fin)
    XBs = [basis(xs[t]) for t in range(P)]

    def dconv_like(XH, T1o, T1i, T2o, T2i, W, b):
        Hm = XH @ W[0, 0] + XH @ W[1, 0]
        Hm = Hm + T1o @ W[0, 1] + T1i @ W[1, 1]
        Hm = Hm + T2o @ W[0, 2] + T2i @ W[1, 2]
        return Hm + b

    H = jnp.zeros((N, Fout), jnp.float32)
    hs = []
    for t in range(P):
        X = xs[t]
        Xf, Xb, Xff, Xbb = XBs[t]
        Hf, Hb, Hff, Hbb = basis(H)
        XH = jnp.concatenate([X, H], axis=-1)
        T1o = jnp.concatenate([Xf, Hf], axis=-1)
        T1i = jnp.concatenate([Xb, Hb], axis=-1)
        T2o = jnp.concatenate([Xff, Hff], axis=-1)
        T2i = jnp.concatenate([Xbb, Hbb], axis=-1)
        Z = jax.nn.sigmoid(dconv_like(XH, T1o, T1i, T2o, T2i, Wz, bz))
        Rg = jax.nn.sigmoid(dconv_like(XH, T1o, T1i, T2o, T2i, Wr, br))
        G = Rg * H
        Gf, Gb, Gff, Gbb = basis(G)
        XG = jnp.concatenate([X, G], axis=-1)
        U1o = jnp.concatenate([Xf, Gf], axis=-1)
        U1i = jnp.concatenate([Xb, Gb], axis=-1)
        U2o = jnp.concatenate([Xff, Gff], axis=-1)
        U2i = jnp.concatenate([Xbb, Gbb], axis=-1)
        Ht = jnp.tanh(dconv_like(XG, U1o, U1i, U2o, U2i, Wh, bh))
        H = jax.nn.relu(Z * H + (1.0 - Z) * Ht)
        hs.append(H)

    Hs = jnp.stack(hs, axis=0)
    out = _proj(Hs.reshape(P * N, Fout), W_lin, b_lin).reshape(P, N, Fin)
    out_seq = jnp.transpose(out[:, :, None, :], (2, 1, 3, 0))
    return out_seq, H


# speedup vs baseline: 4.9063x; 2.1954x over previous
import functools

import jax
import jax.numpy as jnp
from jax import lax
from jax.experimental import pallas as pl
from jax.experimental.pallas import tpu as pltpu
from jax.experimental.pallas import tpu_sc as plsc

_NC = 2
_NS = 16
_NW = _NC * _NS
_F = 128
_CHUNK = 400
_PHASE_E = 4000


_FH = _F // 2


def _lane_bcast(v16, l):
    return lax.gather(
        v16, jnp.full((16, 1), l, jnp.int32),
        lax.GatherDimensionNumbers(offset_dims=(), collapsed_slice_dims=(0,),
                                   start_index_map=(0,)),
        (1,), mode=lax.GatherScatterMode.PROMISE_IN_BOUNDS)


def _prop_body(n_pad, table2, src, dst, scale, out,
               src_a, dst_a, scale_a, rows0, rows1, acc,
               sem0, sem1, ssem0, ssem1):
    c = lax.axis_index("c")
    s = lax.axis_index("s")
    rpt = n_pad // _NS
    e_per_w = src.shape[0] // _NS
    base_w = s * e_per_w
    rows = (rows0, rows1)
    sems = (sem0, sem1)
    ssems = (ssem0, ssem1)

    zvec = jnp.zeros((16,), jnp.float32)

    def zf(r, carry):
        for j in range(_FH // 16):
            rows0[r, pl.ds(j * 16, 16)] = zvec
        return carry

    lax.fori_loop(0, _CHUNK, zf, 0)
    pltpu.sync_copy(rows0, acc.at[pl.ds(s * rpt, _CHUNK)])
    pltpu.sync_copy(rows0.at[pl.ds(0, rpt - _CHUNK)],
                    acc.at[pl.ds(s * rpt + _CHUNK, rpt - _CHUNK)])
    plsc.subcore_barrier()

    def issue_gather(i, b):
        pltpu.async_copy(table2.at[src_a.at[pl.ds(i * _CHUNK, _CHUNK)]],
                         rows[b], sems[b])

    def wait_gather(b):
        pltpu.make_async_copy(table2.at[pl.ds(0, _CHUNK)], rows[b],
                              sems[b]).wait()

    def issue_scatter(i, b):
        pltpu.async_copy(rows[b],
                         acc.at[dst_a.at[pl.ds(i * _CHUNK, _CHUNK)]],
                         ssems[b], add=True)

    def drain_scatter(b):
        pltpu.make_async_copy(table2.at[pl.ds(0, _CHUNK)], rows[b],
                              ssems[b]).wait()

    def scale_rows(i, b):
        @plsc.parallel_loop(0, _CHUNK // 16, unroll=2)
        def grp(g):
            e0 = g * 16
            sv16 = scale_a[pl.ds(i * _CHUNK + e0, 16)]
            for l in range(16):
                bl = _lane_bcast(sv16, l)
                for j in range(_FH // 16):
                    sl = pl.ds(j * 16, 16)
                    rows[b][e0 + l, sl] = rows[b][e0 + l, sl] * bl

    n_ph_pairs = _PHASE_E // _CHUNK // 2

    def phase_body(h, carry):
        pbase = base_w + h * _PHASE_E
        pltpu.sync_copy(src.at[pl.ds(pbase, _PHASE_E)], src_a)
        pltpu.sync_copy(dst.at[pl.ds(pbase, _PHASE_E)], dst_a)
        pltpu.sync_copy(scale.at[pl.ds(pbase, _PHASE_E)], scale_a)

        def xf(g, carry2):
            sl = pl.ds(g * 16, 16)
            src_a[sl] = src_a[sl] * 2 + c
            return carry2

        lax.fori_loop(0, _PHASE_E // 16, xf, 0)
        issue_gather(0, 0)

        def pair_body(k, carry2):
            ia, ib = 2 * k, 2 * k + 1
            wait_gather(0)

            @pl.when(k > 0)
            def _drain_b():
                drain_scatter(1)

            issue_gather(ib, 1)
            scale_rows(ia, 0)
            issue_scatter(ia, 0)
            wait_gather(1)
            scale_rows(ib, 1)
            issue_scatter(ib, 1)

            @pl.when(k < n_ph_pairs - 1)
            def _next_a():
                drain_scatter(0)
                issue_gather(ib + 1, 0)

            return carry2

        lax.fori_loop(0, n_ph_pairs, pair_body, 0)
        drain_scatter(0)
        drain_scatter(1)
        return carry

    lax.fori_loop(0, e_per_w // _PHASE_E, phase_body, 0)
    plsc.subcore_barrier()
    pltpu.sync_copy(acc.at[pl.ds(s * rpt, rpt)],
                    out.at[c, pl.ds(s * rpt, rpt)])


def _sc_prop(table, src, dst, scale, zeros):
    del zeros
    n = table.shape[0]
    n_pad = (n + _NS * 8 - 1) // (_NS * 8) * (_NS * 8)
    e_per_w = src.shape[0] // _NS
    mesh = plsc.VectorSubcoreMesh(core_axis_name="c", subcore_axis_name="s")
    kern = functools.partial(
        pl.kernel,
        mesh=mesh,
        compiler_params=pltpu.CompilerParams(use_tc_tiling_on_sc=False),
        out_type=jax.ShapeDtypeStruct((_NC, n_pad, _FH), jnp.float32),
        scratch_types=[
            pltpu.VMEM((_PHASE_E,), jnp.int32),
            pltpu.VMEM((_PHASE_E,), jnp.int32),
            pltpu.VMEM((_PHASE_E,), jnp.float32),
            pltpu.VMEM((_CHUNK, _FH), jnp.float32),
            pltpu.VMEM((_CHUNK, _FH), jnp.float32),
            pltpu.VMEM_SHARED((n_pad, _FH), jnp.float32),
            pltpu.SemaphoreType.DMA,
            pltpu.SemaphoreType.DMA,
            pltpu.SemaphoreType.DMA,
            pltpu.SemaphoreType.DMA,
        ],
    )(functools.partial(_prop_body, n_pad))
    p = kern(table.reshape(2 * n, _FH), src, dst, scale)
    return p[:, :n].transpose(1, 0, 2).reshape(n, _F)


def _proj_matmul_body(x_ref, w_ref, b_ref, o_ref):
    o_ref[...] = (
        jnp.dot(x_ref[...], w_ref[...], preferred_element_type=jnp.float32)
        + b_ref[...]
    )


def _proj(x2d, W, b):
    M, F = x2d.shape
    Fo = W.shape[1]
    BM = 1000
    return pl.pallas_call(
        _proj_matmul_body,
        grid=(M // BM,),
        in_specs=[
            pl.BlockSpec((BM, F), lambda i: (i, 0)),
            pl.BlockSpec((F, Fo), lambda i: (0, 0)),
            pl.BlockSpec((1, Fo), lambda i: (0, 0)),
        ],
        out_specs=pl.BlockSpec((BM, Fo), lambda i: (i, 0)),
        out_shape=jax.ShapeDtypeStruct((M, Fo), jnp.float32),
    )(x2d, W, b.reshape(1, Fo))


def kernel(x, edge_index, edge_weight, Wz, bz, Wr, br, Wh, bh, W_lin, b_lin):
    B, N, Fin, P = x.shape
    Fout = Wz.shape[-1]
    row, col = edge_index[0], edge_index[1]

    deg_out = jnp.zeros((N,), dtype=edge_weight.dtype).at[row].add(edge_weight)
    deg_in = jnp.zeros((N,), dtype=edge_weight.dtype).at[col].add(edge_weight)
    deg_out_inv = jnp.where(deg_out > 0, 1.0 / jnp.where(deg_out > 0, deg_out, 1.0), 0.0)
    deg_in_inv = jnp.where(deg_in > 0, 1.0 / jnp.where(deg_in > 0, deg_in, 1.0), 0.0)
    norm_out = deg_out_inv[row] * edge_weight
    norm_in = deg_in_inv[col] * edge_weight

    n_pad = (N + _NS * 8 - 1) // (_NS * 8) * (_NS * 8)
    zeros = jnp.zeros((n_pad, _FH), jnp.float32)

    def pf(v):
        return _sc_prop(v, col, row, norm_out, zeros)

    def pb(v):
        return _sc_prop(v, row, col, norm_in, zeros)

    def basis(S):
        Sf = pf(S)
        Sb = pb(S)
        Sff = 2.0 * pf(Sf) - S
        Sbb = 2.0 * pb(Sb) - S
        return Sf, Sb, Sff, Sbb

    xs = jnp.transpose(x, (3, 1, 0, 2)).reshape(P, N, B * Fin)
    XBs = [basis(xs[t]) for t in range(P)]

    def dconv_like(XH, T1o, T1i, T2o, T2i, W, b):
        Hm = XH @ W[0, 0] + XH @ W[1, 0]
        Hm = Hm + T1o @ W[0, 1] + T1i @ W[1, 1]
        Hm = Hm + T2o @ W[0, 2] + T2i @ W[1, 2]
        return Hm + b

    H = jnp.zeros((N, Fout), jnp.float32)
    hs = []
    for t in range(P):
        X = xs[t]
        Xf, Xb, Xff, Xbb = XBs[t]
        Hf, Hb, Hff, Hbb = basis(H)
        XH = jnp.concatenate([X, H], axis=-1)
        T1o = jnp.concatenate([Xf, Hf], axis=-1)
        T1i = jnp.concatenate([Xb, Hb], axis=-1)
        T2o = jnp.concatenate([Xff, Hff], axis=-1)
        T2i = jnp.concatenate([Xbb, Hbb], axis=-1)
        Z = jax.nn.sigmoid(dconv_like(XH, T1o, T1i, T2o, T2i, Wz, bz))
        Rg = jax.nn.sigmoid(dconv_like(XH, T1o, T1i, T2o, T2i, Wr, br))
        G = Rg * H
        Gf, Gb, Gff, Gbb = basis(G)
        XG = jnp.concatenate([X, G], axis=-1)
        U1o = jnp.concatenate([Xf, Gf], axis=-1)
        U1i = jnp.concatenate([Xb, Gb], axis=-1)
        U2o = jnp.concatenate([Xff, Gff], axis=-1)
        U2i = jnp.concatenate([Xbb, Gbb], axis=-1)
        Ht = jnp.tanh(dconv_like(XG, U1o, U1i, U2o, U2i, Wh, bh))
        H = jax.nn.relu(Z * H + (1.0 - Z) * Ht)
        hs.append(H)

    Hs = jnp.stack(hs, axis=0)
    out = _proj(Hs.reshape(P * N, Fout), W_lin, b_lin).reshape(P, N, Fin)
    out_seq = jnp.transpose(out[:, :, None, :], (2, 1, 3, 0))
    return out_seq, H
